# baseline (speedup 1.0000x reference)
import functools

import jax
import jax.numpy as jnp
from jax import lax
from jax.experimental import pallas as pl
from jax.experimental.pallas import tpu as pltpu
from jax.experimental.pallas import tpu_sc as plsc

_CHUNK = 128
_LANES = 16


@functools.cache
def _build_sc_gather(B: int, V: int, D: int):
    info = plsc.get_sparse_core_info()
    nc, ns = info.num_cores, info.num_subcores
    nw = nc * ns
    assert B % (8 * nw) == 0, "batch must split 8-aligned across subcores"
    b_per_w = B // nw
    assert b_per_w % _CHUNK == 0
    n_chunks = b_per_w // _CHUNK

    mesh = plsc.VectorSubcoreMesh(core_axis_name="c", subcore_axis_name="s")

    @functools.partial(
        pl.kernel,
        mesh=mesh,
        out_type=jax.ShapeDtypeStruct((B, D), jnp.float32),
        scratch_types=[
            pltpu.VMEM((b_per_w,), jnp.int32),
            pltpu.VMEM((b_per_w,), jnp.int32),
            pltpu.VMEM((b_per_w // 2, 2 * D), jnp.float32),
            pltpu.VMEM((b_per_w, D), jnp.float32),
            pltpu.SemaphoreType.DMA,
        ],
    )
    def sc_gather(n_id_hbm, tbl2_hbm, out_hbm, idx_v, pidx_v, pair_v, rows_v,
                  sem):
        wid = lax.axis_index("s") * nc + lax.axis_index("c")
        base = wid * b_per_w
        half = b_per_w // 2
        pltpu.sync_copy(n_id_hbm.at[pl.ds(base, b_per_w)], idx_v)

        for j in range(b_per_w // _LANES):
            sl = pl.ds(j * _LANES, _LANES)
            pidx_v[sl] = lax.shift_right_logical(idx_v[sl], 1)

        for h in range(2):
            for k in range(n_chunks // 2):
                pltpu.async_copy(
                    tbl2_hbm.at[pidx_v.at[pl.ds(h * half + k * _CHUNK,
                                                _CHUNK)]],
                    pair_v.at[pl.ds(k * _CHUNK, _CHUNK)], sem)
            pltpu.make_async_copy(tbl2_hbm.at[pl.ds(0, half)], pair_v,
                                  sem).wait()

            def select_block(jb, _, h=h):
                i0 = jb * _LANES
                offv = (idx_v[pl.ds(h * half + i0, _LANES)] & 1) * D
                for lane in range(_LANES):
                    off = offv[lane]
                    for c in range(D // _LANES):
                        rows_v[h * half + i0 + lane,
                               pl.ds(c * _LANES, _LANES)] = (
                            pair_v[i0 + lane,
                                   pl.ds(off + c * _LANES, _LANES)])
                return 0

            lax.fori_loop(0, half // _LANES, select_block, 0)

        pltpu.sync_copy(rows_v, out_hbm.at[pl.ds(base, b_per_w)])

    return sc_gather


def kernel(n_id, emb_table):
    B = n_id.shape[0]
    V, D = emb_table.shape
    tbl2 = emb_table.reshape(V // 2, 2 * D)
    sc_gather = _build_sc_gather(B, V, D)
    return sc_gather(n_id.astype(jnp.int32), tbl2)


# speedup vs baseline: 1.7417x; 1.7417x over previous
import functools

import jax
import jax.numpy as jnp
from jax import lax
from jax.experimental import pallas as pl
from jax.experimental.pallas import tpu as pltpu
from jax.experimental.pallas import tpu_sc as plsc

_LANES = 16


@functools.cache
def _build_sc_gather(B: int, V: int, D: int):
    info = plsc.get_sparse_core_info()
    nc, ns = info.num_cores, info.num_subcores
    nw = nc * ns
    assert B % (8 * nw) == 0, "batch must split 8-aligned across subcores"
    b_per_w = B // nw

    mesh = plsc.VectorSubcoreMesh(core_axis_name="c", subcore_axis_name="s")

    @functools.partial(
        pl.kernel,
        mesh=mesh,
        out_type=jax.ShapeDtypeStruct((B, D), jnp.float32),
        scratch_types=[
            pltpu.VMEM((b_per_w,), jnp.int32),
            pltpu.VMEM((b_per_w, D), jnp.float32),
            pltpu.SemaphoreType.DMA,
        ],
    )
    def sc_gather(n_id_hbm, tbl_hbm, out_hbm, idx_v, rows_v, sem):
        wid = lax.axis_index("s") * nc + lax.axis_index("c")
        base = wid * b_per_w
        pltpu.sync_copy(n_id_hbm.at[pl.ds(base, b_per_w)], idx_v)

        def fetch_block(jb, _):
            vec = idx_v[pl.ds(jb * _LANES, _LANES)]
            for lane in range(_LANES):
                row = vec[lane]
                pltpu.async_copy(
                    tbl_hbm.at[row], rows_v.at[jb * _LANES + lane], sem)
            return 0

        lax.fori_loop(0, b_per_w // _LANES, fetch_block, 0)
        pltpu.make_async_copy(
            tbl_hbm.at[pl.ds(0, b_per_w)], rows_v, sem).wait()

        pltpu.sync_copy(rows_v, out_hbm.at[pl.ds(base, b_per_w)])

    return sc_gather


def kernel(n_id, emb_table):
    B = n_id.shape[0]
    V, D = emb_table.shape
    sc_gather = _build_sc_gather(B, V, D)
    return sc_gather(n_id.astype(jnp.int32), emb_table)
